# Initial kernel scaffold; baseline (speedup 1.0000x reference)
import jax
import jax.numpy as jnp
from jax.experimental import pallas as pl


def kernel(h, edge_index, Wq, Wk, Wv, Wo, bo, ln1_g, ln1_b, W1, b1, W2, b2, ln2_g, ln2_b):
    raise NotImplementedError("write your pallas kernel here")


# speedup vs baseline: 31.5158x; 31.5158x over previous
import jax
import jax.numpy as jnp
from jax import lax
from jax.experimental import pallas as pl
from jax.experimental.pallas import tpu as pltpu
from jax.experimental.pallas import tpu_sc as plsc

_NC = 2
_NS = 16
_NL = 16
_H = 8
_HC = 4
_DH = 16
_B = 80


def _qkv_body(h_ref, w_ref, q_ref, kv_ref):
    acc = jnp.dot(h_ref[...], w_ref[...], preferred_element_type=jnp.float32)
    q = acc[:, :128] * 0.25
    q_ref[0] = q[:, :64]
    q_ref[1] = q[:, 64:]
    kv_ref[0] = jnp.concatenate([acc[:, 128:192], acc[:, 256:320]], axis=1)
    kv_ref[1] = jnp.concatenate([acc[:, 192:256], acc[:, 320:384]], axis=1)


def _qkv_call(h, w_all):
    n = h.shape[0]
    bn = 2000
    return pl.pallas_call(
        _qkv_body,
        grid=(n // bn,),
        in_specs=[
            pl.BlockSpec((bn, 128), lambda i: (i, i * 0)),
            pl.BlockSpec((128, 384), lambda i: (i * 0, i * 0)),
        ],
        out_specs=[
            pl.BlockSpec((_NC, bn, 64), lambda i: (i * 0, i, i * 0)),
            pl.BlockSpec((_NC, bn, 128), lambda i: (i * 0, i, i * 0)),
        ],
        out_shape=[
            jax.ShapeDtypeStruct((_NC, n, 64), jnp.float32),
            jax.ShapeDtypeStruct((_NC, n, 128), jnp.float32),
        ],
    )(h, w_all)


def _edge_call(q_all, kv_all, src, dst):
    n = kv_all.shape[0] // _NC
    e = src.shape[0]
    per_w = e // _NS
    iters = per_w // _B
    n2_pad = -(-((n + 1) // 2) // 128) * 128
    rows_t = n2_pad // _NS
    za_rows = -(-(n * _HC) // 128)
    za_rows = -(-za_rows // 8) * 8
    n2 = n // 2

    mesh = plsc.VectorSubcoreMesh(core_axis_name="c", subcore_axis_name="s")

    def body(q_hbm, kv_hbm, src_hbm, dst_hbm, out_hbm, outz_hbm,
             acc, zsh, zacc, zbuf, idx_s, idx_d, idx_sg, idx_dg, idx_sc,
             kvr, qr, msg, sem1, sem2):
        c = lax.axis_index("c")
        s = lax.axis_index("s")
        lane = lax.iota(jnp.int32, _NL)
        zmask = lane < _HC
        zero16 = jnp.zeros((_NL,), jnp.float32)
        c7 = jnp.full((_NL,), 7, jnp.int32)
        c127 = jnp.full((_NL,), 127, jnp.int32)
        cn = c * n

        for rr in range(8):
            for cc in range(8):
                zbuf[rr, pl.ds(cc * _NL, _NL)] = zero16

        def zc(_, j):
            pltpu.sync_copy(zbuf, acc.at[pl.ds(s * rows_t + j * 8, 8), :])
            return j + 1
        lax.fori_loop(0, rows_t // 8, zc, jnp.int32(0))

        def zz(_, j):
            r = lax.div(j, jnp.int32(8))
            col = lax.rem(j, jnp.int32(8)) * _NL
            zacc[r, pl.ds(col, _NL)] = zero16
            return j + 1
        lax.fori_loop(0, za_rows * 8, zz, jnp.int32(0))

        @pl.when(s < 10)
        def _():
            def zs(_, j):
                pltpu.sync_copy(zbuf, zsh.at[pl.ds(s * 32 + j * 8, 8), :])
                return j + 1
            lax.fori_loop(0, 4, zs, jnp.int32(0))
        plsc.subcore_barrier()

        base0 = s * per_w

        def it(_, i):
            base = pl.multiple_of(base0 + i * _B, 8)
            pltpu.sync_copy(src_hbm.at[pl.ds(base, _B)], idx_s)
            pltpu.sync_copy(dst_hbm.at[pl.ds(base, _B)], idx_d)

            def pp(_2, k):
                sv = idx_s[pl.ds(k * _NL, _NL)]
                dv = idx_d[pl.ds(k * _NL, _NL)]
                idx_sg[pl.ds(k * _NL, _NL)] = sv + cn
                dh = lax.shift_right_logical(dv, jnp.full((_NL,), 1, jnp.int32))
                idx_sc[pl.ds(k * _NL, _NL)] = dh
                idx_dg[pl.ds(k * _NL, _NL)] = dh + c * n2
                return k + 1
            lax.fori_loop(0, _B // _NL, pp, jnp.int32(0))

            d1 = pltpu.async_copy(kv_hbm.at[idx_sg], kvr, sem1)
            d2 = pltpu.async_copy(q_hbm.at[idx_dg], qr, sem2)
            d1.wait()
            d2.wait()

            def grp(_2, k):
                dvec = idx_d[pl.ds(k * _NL, _NL)]
                base_e = k * _NL
                for j in range(_NL):
                    ei = base_e + j
                    d = dvec[j]
                    doff = lax.bitwise_and(d, jnp.int32(1)) * 64
                    ooff = 64 - doff
                    z = zero16
                    for hh in range(_HC):
                        kh = kvr[ei, pl.ds(hh * _DH, _DH)]
                        qh = qr[ei, pl.ds(doff + hh * _DH, _DH)]
                        sc = jnp.sum(kh * qh)
                        sc = jnp.minimum(jnp.maximum(sc, -5.0), 5.0)
                        pe = jnp.exp(jnp.full((_NL,), sc, jnp.float32))
                        vh = kvr[ei, pl.ds(64 + hh * _DH, _DH)]
                        msg[ei, pl.ds(doff + hh * _DH, _DH)] = vh * pe
                        msg[ei, pl.ds(ooff + hh * _DH, _DH)] = zero16
                        z = jnp.where(lane == hh, pe, z)
                    flatv = lane + d * _HC
                    rowv = lax.shift_right_logical(flatv, c7)
                    colv = lax.bitwise_and(flatv, c127)
                    plsc.addupdate_scatter(zacc, [rowv, colv], z, mask=zmask)
                return k + 1
            lax.fori_loop(0, _B // _NL, grp, jnp.int32(0))

            pltpu.sync_copy(msg, acc.at[idx_sc], add=True)
            return i + 1
        lax.fori_loop(0, iters, it, jnp.int32(0))

        for ch in range(za_rows // _B):
            for k in range(_B // _NL):
                idx_s[pl.ds(k * _NL, _NL)] = lane + (ch * _B + k * _NL)
            pltpu.sync_copy(zacc.at[pl.ds(ch * _B, _B), :],
                            zsh.at[idx_s], add=True)
        plsc.subcore_barrier()
        pltpu.sync_copy(acc.at[pl.ds(s * rows_t, rows_t), :],
                        out_hbm.at[c, pl.ds(s * rows_t, rows_t), :])

        @pl.when(s < 10)
        def _():
            pltpu.sync_copy(zsh.at[pl.ds(s * 32, 32), :],
                            outz_hbm.at[c, pl.ds(s * 32, 32), :])

    fn = pl.kernel(
        body,
        out_type=[
            jax.ShapeDtypeStruct((_NC, n2_pad, 128), jnp.float32),
            jax.ShapeDtypeStruct((_NC, za_rows, 128), jnp.float32),
        ],
        mesh=mesh,
        compiler_params=pltpu.CompilerParams(needs_layout_passes=False),
        scratch_types=[
            pltpu.VMEM_SHARED((n2_pad, 128), jnp.float32),
            pltpu.VMEM_SHARED((za_rows, 128), jnp.float32),
            pltpu.VMEM((za_rows, 128), jnp.float32),
            pltpu.VMEM((8, 128), jnp.float32),
            pltpu.VMEM((_B,), jnp.int32),
            pltpu.VMEM((_B,), jnp.int32),
            pltpu.VMEM((_B,), jnp.int32),
            pltpu.VMEM((_B,), jnp.int32),
            pltpu.VMEM((_B,), jnp.int32),
            pltpu.VMEM((_B, 128), jnp.float32),
            pltpu.VMEM((_B, 128), jnp.float32),
            pltpu.VMEM((_B, 128), jnp.float32),
            pltpu.SemaphoreType.DMA,
            pltpu.SemaphoreType.DMA,
        ],
    )
    wv, zp = fn(q_all, kv_all, src, dst)
    wv = wv.reshape(_NC, 2 * n2_pad, 64)
    zp = zp.reshape(_NC, za_rows * 128 // _HC, _HC)
    return wv, zp


def _ln(x, g, b):
    mu = jnp.mean(x, axis=1, keepdims=True)
    xc = x - mu
    var = jnp.mean(xc * xc, axis=1, keepdims=True)
    return xc * lax.rsqrt(var + 1e-5) * g + b


def _epi_body(h_ref, p_ref, z_ref, wo_ref, bo_ref, g1_ref, b1_ref,
              w1_ref, c1_ref, w2_ref, c2_ref, g2_ref, b2_ref, o_ref):
    wv = jnp.concatenate([p_ref[0], p_ref[1]], axis=1)
    z8 = jnp.concatenate([z_ref[0], z_ref[1]], axis=1)
    ii = lax.broadcasted_iota(jnp.int32, (_H, 128), 1)
    jj = lax.broadcasted_iota(jnp.int32, (_H, 128), 0) * _DH
    r_sel = ((ii >= jj) & (ii < jj + _DH)).astype(jnp.float32)
    zr = jnp.dot(z8, r_sel, preferred_element_type=jnp.float32)
    ha = wv / (zr + 1e-6)
    h2 = jnp.dot(ha, wo_ref[...], preferred_element_type=jnp.float32) + bo_ref[...]
    r1 = h_ref[...] + h2
    n1 = _ln(r1, g1_ref[...], b1_ref[...])
    f = jnp.dot(n1, w1_ref[...], preferred_element_type=jnp.float32) + c1_ref[...]
    f = jnp.maximum(f, 0.0)
    f = jnp.dot(f, w2_ref[...], preferred_element_type=jnp.float32) + c2_ref[...]
    r2 = n1 + f
    o_ref[...] = _ln(r2, g2_ref[...], b2_ref[...])


def _epi_call(h, wv, zp, Wo, bo, ln1_g, ln1_b, W1, b1, W2, b2, ln2_g, ln2_b):
    n = h.shape[0]
    bn = 2000
    full = lambda i: (i * 0, i * 0)
    return pl.pallas_call(
        _epi_body,
        grid=(n // bn,),
        in_specs=[
            pl.BlockSpec((bn, 128), lambda i: (i, i * 0)),
            pl.BlockSpec((_NC, bn, 64), lambda i: (i * 0, i, i * 0)),
            pl.BlockSpec((_NC, bn, _HC), lambda i: (i * 0, i, i * 0)),
            pl.BlockSpec((128, 128), full),
            pl.BlockSpec((1, 128), full),
            pl.BlockSpec((1, 128), full),
            pl.BlockSpec((1, 128), full),
            pl.BlockSpec((128, 256), full),
            pl.BlockSpec((1, 256), full),
            pl.BlockSpec((256, 128), full),
            pl.BlockSpec((1, 128), full),
            pl.BlockSpec((1, 128), full),
            pl.BlockSpec((1, 128), full),
        ],
        out_specs=pl.BlockSpec((bn, 128), lambda i: (i, i * 0)),
        out_shape=jax.ShapeDtypeStruct((n, 128), jnp.float32),
    )(h, wv, zp, Wo, bo.reshape(1, -1), ln1_g.reshape(1, -1),
      ln1_b.reshape(1, -1), W1, b1.reshape(1, -1), W2, b2.reshape(1, -1),
      ln2_g.reshape(1, -1), ln2_b.reshape(1, -1))


def kernel(h, edge_index, Wq, Wk, Wv, Wo, bo, ln1_g, ln1_b,
           W1, b1, W2, b2, ln2_g, ln2_b):
    f32 = jnp.float32
    h, Wq, Wk, Wv, Wo, bo, ln1_g, ln1_b, W1, b1, W2, b2, ln2_g, ln2_b = (
        x.astype(f32) for x in
        (h, Wq, Wk, Wv, Wo, bo, ln1_g, ln1_b, W1, b1, W2, b2, ln2_g, ln2_b))
    src = edge_index[0].astype(jnp.int32)
    dst = edge_index[1].astype(jnp.int32)
    w_all = jnp.concatenate([Wq, Wk, Wv], axis=1)
    q_all, kv_all = _qkv_call(h, w_all)
    n = h.shape[0]
    wv, zp = _edge_call(q_all.reshape(_NC * (n // 2), 128),
                        kv_all.reshape(_NC * n, 128), src, dst)
    out = _epi_call(h, wv, zp, Wo, bo, ln1_g, ln1_b,
                    W1, b1, W2, b2, ln2_g, ln2_b)
    return out.astype(jnp.float64)


# speedup vs baseline: 33.0680x; 1.0493x over previous
import jax
import jax.numpy as jnp
from jax import lax
from jax.experimental import pallas as pl
from jax.experimental.pallas import tpu as pltpu
from jax.experimental.pallas import tpu_sc as plsc

_NC = 2
_NS = 16
_NL = 16
_H = 8
_HC = 4
_DH = 16
_B = 80
_CH = 10


def _qkv_body(h_ref, w_ref, q_ref, kv_ref):
    acc = jnp.dot(h_ref[...], w_ref[...], preferred_element_type=jnp.float32)
    q = acc[:, :128] * 0.25
    q_ref[0] = q[:, :64]
    q_ref[1] = q[:, 64:]
    kv_ref[0] = jnp.concatenate([acc[:, 128:192], acc[:, 256:320]], axis=1)
    kv_ref[1] = jnp.concatenate([acc[:, 192:256], acc[:, 320:384]], axis=1)


def _qkv_call(h, w_all):
    n = h.shape[0]
    bn = 2000
    return pl.pallas_call(
        _qkv_body,
        grid=(n // bn,),
        in_specs=[
            pl.BlockSpec((bn, 128), lambda i: (i, i * 0)),
            pl.BlockSpec((128, 384), lambda i: (i * 0, i * 0)),
        ],
        out_specs=[
            pl.BlockSpec((_NC, bn, 64), lambda i: (i * 0, i, i * 0)),
            pl.BlockSpec((_NC, bn, 128), lambda i: (i * 0, i, i * 0)),
        ],
        out_shape=[
            jax.ShapeDtypeStruct((_NC, n, 64), jnp.float32),
            jax.ShapeDtypeStruct((_NC, n, 128), jnp.float32),
        ],
    )(h, w_all)


def _edge_call(q_all, kv_all, src, dst):
    n = kv_all.shape[0] // _NC
    e = src.shape[0]
    per_w = e // _NS
    chunk_e = _B * _CH
    n_chunks = per_w // chunk_e
    n2_pad = -(-((n + 1) // 2) // 128) * 128
    rows_t = n2_pad // _NS
    zrows = -(-(-(-n // 16)) // 128) * 128
    n2 = n // 2

    mesh = plsc.VectorSubcoreMesh(core_axis_name="c", subcore_axis_name="s")

    def body(q_hbm, kv_hbm, src_hbm, dst_hbm, out_hbm, outz_hbm,
             acc, zpk, zbuf, raw_s, raw_d, idx_sg, idx_dg, idx_sc, idx_dz,
             kvr, qr, msg, zmsg, gsem_kv0, gsem_kv1, gsem_q0, gsem_q1):
        gsem_kv = (gsem_kv0, gsem_kv1)
        gsem_q = (gsem_q0, gsem_q1)
        c = lax.axis_index("c")
        s = lax.axis_index("s")
        lane = lax.iota(jnp.int32, _NL)
        zmask = lane < _HC
        zero16 = jnp.zeros((_NL,), jnp.float32)
        cn = c * n
        c4 = c * _HC

        for rr in range(8):
            for cc in range(8):
                zbuf[rr, pl.ds(cc * _NL, _NL)] = zero16

        def zc(_, j):
            pltpu.sync_copy(zbuf, acc.at[pl.ds(s * rows_t + j * 8, 8), :])
            return j + 1
        lax.fori_loop(0, rows_t // 8, zc, jnp.int32(0))

        def zs(_, j):
            pltpu.sync_copy(zbuf,
                            zpk.at[pl.ds(s * (zrows // _NS) + j * 8, 8), :])
            return j + 1
        lax.fori_loop(0, zrows // _NS // 8, zs, jnp.int32(0))
        plsc.subcore_barrier()

        base0 = s * per_w

        def prep(b, k):
            def pp(_2, kk):
                sv = raw_s[pl.ds(b * _B + kk * _NL, _NL)]
                dv = raw_d[pl.ds(b * _B + kk * _NL, _NL)]
                idx_sg[k, pl.ds(kk * _NL, _NL)] = sv + cn
                dh = lax.shift_right_logical(
                    dv, jnp.full((_NL,), 1, jnp.int32))
                idx_sc[k, pl.ds(kk * _NL, _NL)] = dh
                idx_dg[k, pl.ds(kk * _NL, _NL)] = dh + c * n2
                idx_dz[k, pl.ds(kk * _NL, _NL)] = lax.shift_right_logical(
                    dv, jnp.full((_NL,), 4, jnp.int32))
                return kk + 1
            lax.fori_loop(0, _B // _NL, pp, jnp.int32(0))
            pltpu.async_copy(kv_hbm.at[idx_sg.at[jnp.int32(k)]],
                             kvr.at[jnp.int32(k)], gsem_kv[k])
            pltpu.async_copy(q_hbm.at[idx_dg.at[jnp.int32(k)]],
                             qr.at[jnp.int32(k)], gsem_q[k])

        def compute(b, k, mk):
            pltpu.make_async_copy(kv_hbm.at[idx_sg.at[jnp.int32(k)]],
                                  kvr.at[jnp.int32(k)], gsem_kv[k]).wait()
            pltpu.make_async_copy(q_hbm.at[idx_dg.at[jnp.int32(k)]],
                                  qr.at[jnp.int32(k)], gsem_q[k]).wait()

            def grp(_2, kk):
                dvec = raw_d[pl.ds(b * _B + kk * _NL, _NL)]
                base_e = kk * _NL
                for j in range(_NL):
                    ei = base_e + j
                    d = dvec[j]
                    doff = lax.bitwise_and(d, jnp.int32(1)) * 64
                    ooff = 64 - doff
                    bcol = lax.bitwise_and(d, jnp.int32(15)) * 8 + c4
                    goff = lax.bitwise_and(bcol, jnp.int32(15))
                    col16 = bcol - goff
                    z = zero16
                    for hh in range(_HC):
                        kh = kvr[k, ei, pl.ds(hh * _DH, _DH)]
                        qh = qr[k, ei, pl.ds(doff + hh * _DH, _DH)]
                        sc = jnp.sum(kh * qh)
                        sc = jnp.minimum(jnp.maximum(sc, -5.0), 5.0)
                        pe = jnp.exp(jnp.full((_NL,), sc, jnp.float32))
                        vh = kvr[k, ei, pl.ds(64 + hh * _DH, _DH)]
                        msg[ei, pl.ds(doff + hh * _DH, _DH)] = vh * pe
                        msg[ei, pl.ds(ooff + hh * _DH, _DH)] = zero16
                        z = jnp.where(lane == hh + goff, pe, z)
                    for g in range(8):
                        zmsg[ei, pl.ds(g * _NL, _NL)] = zero16
                    zmsg[ei, pl.ds(col16, _NL)] = z
                return kk + 1
            lax.fori_loop(0, _B // _NL, grp, jnp.int32(0))
            pltpu.sync_copy(msg, acc.at[idx_sc.at[jnp.int32(k)]], add=True)
            pltpu.sync_copy(zmsg, zpk.at[idx_dz.at[jnp.int32(k)]], add=True)

        def chunk(_, ci):
            base_c = pl.multiple_of(base0 + ci * chunk_e, 8)
            pltpu.sync_copy(src_hbm.at[pl.ds(base_c, chunk_e)], raw_s)
            pltpu.sync_copy(dst_hbm.at[pl.ds(base_c, chunk_e)], raw_d)
            prep(jnp.int32(0), 0)

            def inner(_2, io):
                b0 = io * 2
                b1 = b0 + 1
                prep(b1, 1)
                compute(b0, 0, 0)
                prep(b1 + 1, 0)
                compute(b1, 1, 1)
                return io + 1
            lax.fori_loop(0, _CH // 2 - 1, inner, jnp.int32(0))
            last = jnp.int32(_CH - 2)
            prep(last + 1, 1)
            compute(last, 0, 0)
            compute(last + 1, 1, 1)
            return ci + 1
        lax.fori_loop(0, n_chunks, chunk, jnp.int32(0))

        plsc.subcore_barrier()
        pltpu.sync_copy(acc.at[pl.ds(s * rows_t, rows_t), :],
                        out_hbm.at[c, pl.ds(s * rows_t, rows_t), :])
        zr_t = zrows // _NS
        pltpu.sync_copy(zpk.at[pl.ds(s * zr_t, zr_t), :],
                        outz_hbm.at[c, pl.ds(s * zr_t, zr_t), :])

    fn = pl.kernel(
        body,
        out_type=[
            jax.ShapeDtypeStruct((_NC, n2_pad, 128), jnp.float32),
            jax.ShapeDtypeStruct((_NC, zrows, 128), jnp.float32),
        ],
        mesh=mesh,
        compiler_params=pltpu.CompilerParams(needs_layout_passes=False),
        scratch_types=[
            pltpu.VMEM_SHARED((n2_pad, 128), jnp.float32),
            pltpu.VMEM_SHARED((zrows, 128), jnp.float32),
            pltpu.VMEM((8, 128), jnp.float32),
            pltpu.VMEM((_B * _CH,), jnp.int32),
            pltpu.VMEM((_B * _CH,), jnp.int32),
            pltpu.VMEM((2, _B), jnp.int32),
            pltpu.VMEM((2, _B), jnp.int32),
            pltpu.VMEM((2, _B), jnp.int32),
            pltpu.VMEM((2, _B), jnp.int32),
            pltpu.VMEM((2, _B, 128), jnp.float32),
            pltpu.VMEM((2, _B, 128), jnp.float32),
            pltpu.VMEM((_B, 128), jnp.float32),
            pltpu.VMEM((_B, 128), jnp.float32),
            pltpu.SemaphoreType.DMA,
            pltpu.SemaphoreType.DMA,
            pltpu.SemaphoreType.DMA,
            pltpu.SemaphoreType.DMA,
        ],
    )
    wv, zp = fn(q_all, kv_all, src, dst)
    wv = wv.reshape(_NC, 2 * n2_pad, 64)
    zp = zp.reshape(_NC, zrows * 16, _H)
    return wv, zp


def _ln(x, g, b):
    mu = jnp.mean(x, axis=1, keepdims=True)
    xc = x - mu
    var = jnp.mean(xc * xc, axis=1, keepdims=True)
    return xc * lax.rsqrt(var + 1e-5) * g + b


def _epi_body(h_ref, p_ref, z_ref, wo_ref, bo_ref, g1_ref, b1_ref,
              w1_ref, c1_ref, w2_ref, c2_ref, g2_ref, b2_ref, o_ref):
    wv = jnp.concatenate([p_ref[0], p_ref[1]], axis=1)
    z8 = z_ref[0] + z_ref[1]
    ii = lax.broadcasted_iota(jnp.int32, (_H, 128), 1)
    jj = lax.broadcasted_iota(jnp.int32, (_H, 128), 0) * _DH
    r_sel = ((ii >= jj) & (ii < jj + _DH)).astype(jnp.float32)
    zr = jnp.dot(z8, r_sel, preferred_element_type=jnp.float32)
    ha = wv / (zr + 1e-6)
    h2 = jnp.dot(ha, wo_ref[...], preferred_element_type=jnp.float32) + bo_ref[...]
    r1 = h_ref[...] + h2
    n1 = _ln(r1, g1_ref[...], b1_ref[...])
    f = jnp.dot(n1, w1_ref[...], preferred_element_type=jnp.float32) + c1_ref[...]
    f = jnp.maximum(f, 0.0)
    f = jnp.dot(f, w2_ref[...], preferred_element_type=jnp.float32) + c2_ref[...]
    r2 = n1 + f
    o_ref[...] = _ln(r2, g2_ref[...], b2_ref[...])


def _epi_call(h, wv, zp, Wo, bo, ln1_g, ln1_b, W1, b1, W2, b2, ln2_g, ln2_b):
    n = h.shape[0]
    bn = 2000
    full = lambda i: (i * 0, i * 0)
    return pl.pallas_call(
        _epi_body,
        grid=(n // bn,),
        in_specs=[
            pl.BlockSpec((bn, 128), lambda i: (i, i * 0)),
            pl.BlockSpec((_NC, bn, 64), lambda i: (i * 0, i, i * 0)),
            pl.BlockSpec((_NC, bn, _H), lambda i: (i * 0, i, i * 0)),
            pl.BlockSpec((128, 128), full),
            pl.BlockSpec((1, 128), full),
            pl.BlockSpec((1, 128), full),
            pl.BlockSpec((1, 128), full),
            pl.BlockSpec((128, 256), full),
            pl.BlockSpec((1, 256), full),
            pl.BlockSpec((256, 128), full),
            pl.BlockSpec((1, 128), full),
            pl.BlockSpec((1, 128), full),
            pl.BlockSpec((1, 128), full),
        ],
        out_specs=pl.BlockSpec((bn, 128), lambda i: (i, i * 0)),
        out_shape=jax.ShapeDtypeStruct((n, 128), jnp.float32),
    )(h, wv, zp, Wo, bo.reshape(1, -1), ln1_g.reshape(1, -1),
      ln1_b.reshape(1, -1), W1, b1.reshape(1, -1), W2, b2.reshape(1, -1),
      ln2_g.reshape(1, -1), ln2_b.reshape(1, -1))


def kernel(h, edge_index, Wq, Wk, Wv, Wo, bo, ln1_g, ln1_b,
           W1, b1, W2, b2, ln2_g, ln2_b):
    f32 = jnp.float32
    h, Wq, Wk, Wv, Wo, bo, ln1_g, ln1_b, W1, b1, W2, b2, ln2_g, ln2_b = (
        x.astype(f32) for x in
        (h, Wq, Wk, Wv, Wo, bo, ln1_g, ln1_b, W1, b1, W2, b2, ln2_g, ln2_b))
    src = edge_index[0].astype(jnp.int32)
    dst = edge_index[1].astype(jnp.int32)
    w_all = jnp.concatenate([Wq, Wk, Wv], axis=1)
    q_all, kv_all = _qkv_call(h, w_all)
    n = h.shape[0]
    wv, zp = _edge_call(q_all.reshape(_NC * (n // 2), 128),
                        kv_all.reshape(_NC * n, 128), src, dst)
    out = _epi_call(h, wv, zp, Wo, bo, ln1_g, ln1_b,
                    W1, b1, W2, b2, ln2_g, ln2_b)
    return out.astype(jnp.float64)


# speedup vs baseline: 34.5718x; 1.0455x over previous
import jax
import jax.numpy as jnp
from jax import lax
from jax.experimental import pallas as pl
from jax.experimental.pallas import tpu as pltpu
from jax.experimental.pallas import tpu_sc as plsc

_NC = 2
_NS = 16
_NL = 16
_H = 8
_HC = 4
_DH = 16
_B = 80
_CH = 10


def _qkv_body(h_ref, w_ref, q_ref, kv_ref):
    acc = jnp.dot(h_ref[...], w_ref[...], preferred_element_type=jnp.float32)
    q = acc[:, :128] * 0.25
    q_ref[0] = q[:, :64]
    q_ref[1] = q[:, 64:]
    kv_ref[0] = jnp.concatenate([acc[:, 128:192], acc[:, 256:320]], axis=1)
    kv_ref[1] = jnp.concatenate([acc[:, 192:256], acc[:, 320:384]], axis=1)


def _qkv_call(h, w_all):
    n = h.shape[0]
    bn = 2000
    return pl.pallas_call(
        _qkv_body,
        grid=(n // bn,),
        in_specs=[
            pl.BlockSpec((bn, 128), lambda i: (i, i * 0)),
            pl.BlockSpec((128, 384), lambda i: (i * 0, i * 0)),
        ],
        out_specs=[
            pl.BlockSpec((_NC, bn, 64), lambda i: (i * 0, i, i * 0)),
            pl.BlockSpec((_NC, bn, 128), lambda i: (i * 0, i, i * 0)),
        ],
        out_shape=[
            jax.ShapeDtypeStruct((_NC, n, 64), jnp.float32),
            jax.ShapeDtypeStruct((_NC, n, 128), jnp.float32),
        ],
    )(h, w_all)


def _edge_call(q_all, kv_all, src, dst):
    n = kv_all.shape[0] // _NC
    e = src.shape[0]
    per_w = e // _NS
    chunk_e = _B * _CH
    n_chunks = per_w // chunk_e
    n2_pad = -(-((n + 1) // 2) // 128) * 128
    rows_t = n2_pad // _NS
    za_rows = -(-(-(-(n * _HC) // 128)) // 8) * 8
    n2 = n // 2

    mesh = plsc.VectorSubcoreMesh(core_axis_name="c", subcore_axis_name="s")

    def body(q_hbm, kv_hbm, src_hbm, dst_hbm, out_hbm, outz_hbm,
             acc, zsh, zacc, zbuf, raw_s, raw_d, idx_sg, idx_dg, idx_sc,
             kvr, qr, msg, gsem_kv0, gsem_kv1, gsem_q):
        gsem_kv = (gsem_kv0, gsem_kv1)
        c = lax.axis_index("c")
        s = lax.axis_index("s")
        lane = lax.iota(jnp.int32, _NL)
        zmask = lane < _HC
        zero16 = jnp.zeros((_NL,), jnp.float32)
        cn = c * n
        c7 = jnp.full((_NL,), 7, jnp.int32)
        c127 = jnp.full((_NL,), 127, jnp.int32)

        for rr in range(8):
            for cc in range(8):
                zbuf[rr, pl.ds(cc * _NL, _NL)] = zero16

        def zc(_, j):
            pltpu.sync_copy(zbuf, acc.at[pl.ds(s * rows_t + j * 8, 8), :])
            return j + 1
        lax.fori_loop(0, rows_t // 8, zc, jnp.int32(0))

        def zz(_, j):
            r = lax.div(j, jnp.int32(8))
            col = lax.rem(j, jnp.int32(8)) * _NL
            zacc[r, pl.ds(col, _NL)] = zero16
            return j + 1
        lax.fori_loop(0, za_rows * 8, zz, jnp.int32(0))

        @pl.when(s < 10)
        def _():
            def zs(_, j):
                pltpu.sync_copy(zbuf, zsh.at[pl.ds(s * 32 + j * 8, 8), :])
                return j + 1
            lax.fori_loop(0, 4, zs, jnp.int32(0))
        plsc.subcore_barrier()

        base0 = s * per_w

        def prep(b, k):
            def pp(_2, kk):
                sv = raw_s[pl.ds(b * _B + kk * _NL, _NL)]
                dv = raw_d[pl.ds(b * _B + kk * _NL, _NL)]
                idx_sg[k, pl.ds(kk * _NL, _NL)] = sv + cn
                dh = lax.shift_right_logical(
                    dv, jnp.full((_NL,), 1, jnp.int32))
                idx_sc[k, pl.ds(kk * _NL, _NL)] = dh
                idx_dg[k, pl.ds(kk * _NL, _NL)] = dh + c * n2
                return kk + 1
            lax.fori_loop(0, _B // _NL, pp, jnp.int32(0))
            pltpu.async_copy(kv_hbm.at[idx_sg.at[jnp.int32(k)]],
                             kvr.at[jnp.int32(k)], gsem_kv[k])

        def fire_q(k):
            pltpu.async_copy(q_hbm.at[idx_dg.at[jnp.int32(k)]], qr, gsem_q)

        def compute(b, k, fq):
            pltpu.make_async_copy(kv_hbm.at[idx_sg.at[jnp.int32(k)]],
                                  kvr.at[jnp.int32(k)], gsem_kv[k]).wait()
            pltpu.make_async_copy(q_hbm.at[idx_dg.at[jnp.int32(k)]], qr,
                                  gsem_q).wait()

            def grp(_2, kk):
                dvec = raw_d[pl.ds(b * _B + kk * _NL, _NL)]
                base_e = kk * _NL
                for j in range(_NL):
                    ei = base_e + j
                    d = dvec[j]
                    doff = lax.bitwise_and(d, jnp.int32(1)) * 64
                    ooff = 64 - doff
                    z = zero16
                    for hh in range(_HC):
                        kh = kvr[k, ei, pl.ds(hh * _DH, _DH)]
                        qh = qr[ei, pl.ds(doff + hh * _DH, _DH)]
                        sc = jnp.sum(kh * qh)
                        sc = jnp.minimum(jnp.maximum(sc, -5.0), 5.0)
                        pe = jnp.exp(jnp.full((_NL,), sc, jnp.float32))
                        vh = kvr[k, ei, pl.ds(64 + hh * _DH, _DH)]
                        msg[ei, pl.ds(doff + hh * _DH, _DH)] = vh * pe
                        msg[ei, pl.ds(ooff + hh * _DH, _DH)] = zero16
                        z = jnp.where(lane == hh, pe, z)
                    flatv = lane + d * _HC
                    rowv = lax.shift_right_logical(flatv, c7)
                    colv = lax.bitwise_and(flatv, c127)
                    plsc.addupdate_scatter(zacc, [rowv, colv], z, mask=zmask)
                return kk + 1
            lax.fori_loop(0, _B // _NL, grp, jnp.int32(0))
            if fq is not None:
                fire_q(fq)
            pltpu.sync_copy(msg, acc.at[idx_sc.at[jnp.int32(k)]], add=True)

        def chunk(_, ci):
            base_c = pl.multiple_of(base0 + ci * chunk_e, 8)
            pltpu.sync_copy(src_hbm.at[pl.ds(base_c, chunk_e)], raw_s)
            pltpu.sync_copy(dst_hbm.at[pl.ds(base_c, chunk_e)], raw_d)
            prep(jnp.int32(0), 0)
            fire_q(0)

            def inner(_2, io):
                b0 = io * 2
                b1 = b0 + 1
                prep(b1, 1)
                compute(b0, 0, 1)
                prep(b1 + 1, 0)
                compute(b1, 1, 0)
                return io + 1
            lax.fori_loop(0, _CH // 2 - 1, inner, jnp.int32(0))
            last = jnp.int32(_CH - 2)
            prep(last + 1, 1)
            compute(last, 0, 1)
            compute(last + 1, 1, None)
            return ci + 1
        lax.fori_loop(0, n_chunks, chunk, jnp.int32(0))

        for ch in range(za_rows // _B):
            for kk in range(_B // _NL):
                idx_sc[0, pl.ds(kk * _NL, _NL)] = lane + (ch * _B + kk * _NL)
            pltpu.sync_copy(zacc.at[pl.ds(ch * _B, _B), :],
                            zsh.at[idx_sc.at[jnp.int32(0)]], add=True)
        plsc.subcore_barrier()
        pltpu.sync_copy(acc.at[pl.ds(s * rows_t, rows_t), :],
                        out_hbm.at[c, pl.ds(s * rows_t, rows_t), :])

        @pl.when(s < 10)
        def _():
            pltpu.sync_copy(zsh.at[pl.ds(s * 32, 32), :],
                            outz_hbm.at[c, pl.ds(s * 32, 32), :])

    fn = pl.kernel(
        body,
        out_type=[
            jax.ShapeDtypeStruct((_NC, n2_pad, 128), jnp.float32),
            jax.ShapeDtypeStruct((_NC, za_rows, 128), jnp.float32),
        ],
        mesh=mesh,
        compiler_params=pltpu.CompilerParams(needs_layout_passes=False),
        scratch_types=[
            pltpu.VMEM_SHARED((n2_pad, 128), jnp.float32),
            pltpu.VMEM_SHARED((za_rows, 128), jnp.float32),
            pltpu.VMEM((za_rows, 128), jnp.float32),
            pltpu.VMEM((8, 128), jnp.float32),
            pltpu.VMEM((_B * _CH,), jnp.int32),
            pltpu.VMEM((_B * _CH,), jnp.int32),
            pltpu.VMEM((2, _B), jnp.int32),
            pltpu.VMEM((2, _B), jnp.int32),
            pltpu.VMEM((2, _B), jnp.int32),
            pltpu.VMEM((2, _B, 128), jnp.float32),
            pltpu.VMEM((_B, 128), jnp.float32),
            pltpu.VMEM((_B, 128), jnp.float32),
            pltpu.SemaphoreType.DMA,
            pltpu.SemaphoreType.DMA,
            pltpu.SemaphoreType.DMA,
        ],
    )
    wv, zp = fn(q_all, kv_all, src, dst)
    wv = wv.reshape(_NC, 2 * n2_pad, 64)
    zp = zp.reshape(_NC, za_rows * 128 // _HC, _HC)
    return wv, zp


def _ln(x, g, b):
    mu = jnp.mean(x, axis=1, keepdims=True)
    xc = x - mu
    var = jnp.mean(xc * xc, axis=1, keepdims=True)
    return xc * lax.rsqrt(var + 1e-5) * g + b


def _epi_body(h_ref, p_ref, z_ref, wo_ref, bo_ref, g1_ref, b1_ref,
              w1_ref, c1_ref, w2_ref, c2_ref, g2_ref, b2_ref, o_ref):
    wv = jnp.concatenate([p_ref[0], p_ref[1]], axis=1)
    z8 = jnp.concatenate([z_ref[0], z_ref[1]], axis=1)
    ii = lax.broadcasted_iota(jnp.int32, (_H, 128), 1)
    jj = lax.broadcasted_iota(jnp.int32, (_H, 128), 0) * _DH
    r_sel = ((ii >= jj) & (ii < jj + _DH)).astype(jnp.float32)
    zr = jnp.dot(z8, r_sel, preferred_element_type=jnp.float32)
    ha = wv / (zr + 1e-6)
    h2 = jnp.dot(ha, wo_ref[...], preferred_element_type=jnp.float32) + bo_ref[...]
    r1 = h_ref[...] + h2
    n1 = _ln(r1, g1_ref[...], b1_ref[...])
    f = jnp.dot(n1, w1_ref[...], preferred_element_type=jnp.float32) + c1_ref[...]
    f = jnp.maximum(f, 0.0)
    f = jnp.dot(f, w2_ref[...], preferred_element_type=jnp.float32) + c2_ref[...]
    r2 = n1 + f
    o_ref[...] = _ln(r2, g2_ref[...], b2_ref[...])


def _epi_call(h, wv, zp, Wo, bo, ln1_g, ln1_b, W1, b1, W2, b2, ln2_g, ln2_b):
    n = h.shape[0]
    bn = 2000
    full = lambda i: (i * 0, i * 0)
    return pl.pallas_call(
        _epi_body,
        grid=(n // bn,),
        in_specs=[
            pl.BlockSpec((bn, 128), lambda i: (i, i * 0)),
            pl.BlockSpec((_NC, bn, 64), lambda i: (i * 0, i, i * 0)),
            pl.BlockSpec((_NC, bn, _HC), lambda i: (i * 0, i, i * 0)),
            pl.BlockSpec((128, 128), full),
            pl.BlockSpec((1, 128), full),
            pl.BlockSpec((1, 128), full),
            pl.BlockSpec((1, 128), full),
            pl.BlockSpec((128, 256), full),
            pl.BlockSpec((1, 256), full),
            pl.BlockSpec((256, 128), full),
            pl.BlockSpec((1, 128), full),
            pl.BlockSpec((1, 128), full),
            pl.BlockSpec((1, 128), full),
        ],
        out_specs=pl.BlockSpec((bn, 128), lambda i: (i, i * 0)),
        out_shape=jax.ShapeDtypeStruct((n, 128), jnp.float32),
    )(h, wv, zp, Wo, bo.reshape(1, -1), ln1_g.reshape(1, -1),
      ln1_b.reshape(1, -1), W1, b1.reshape(1, -1), W2, b2.reshape(1, -1),
      ln2_g.reshape(1, -1), ln2_b.reshape(1, -1))


def kernel(h, edge_index, Wq, Wk, Wv, Wo, bo, ln1_g, ln1_b,
           W1, b1, W2, b2, ln2_g, ln2_b):
    f32 = jnp.float32
    h, Wq, Wk, Wv, Wo, bo, ln1_g, ln1_b, W1, b1, W2, b2, ln2_g, ln2_b = (
        x.astype(f32) for x in
        (h, Wq, Wk, Wv, Wo, bo, ln1_g, ln1_b, W1, b1, W2, b2, ln2_g, ln2_b))
    src = edge_index[0].astype(jnp.int32)
    dst = edge_index[1].astype(jnp.int32)
    w_all = jnp.concatenate([Wq, Wk, Wv], axis=1)
    q_all, kv_all = _qkv_call(h, w_all)
    n = h.shape[0]
    wv, zp = _edge_call(q_all.reshape(_NC * (n // 2), 128),
                        kv_all.reshape(_NC * n, 128), src, dst)
    out = _epi_call(h, wv, zp, Wo, bo, ln1_g, ln1_b,
                    W1, b1, W2, b2, ln2_g, ln2_b)
    return out.astype(jnp.float64)


# speedup vs baseline: 94.7573x; 2.7409x over previous
import jax
import jax.numpy as jnp
from jax import lax
from jax.experimental import pallas as pl
from jax.experimental.pallas import tpu as pltpu
from jax.experimental.pallas import tpu_sc as plsc

_NC = 2
_NS = 16
_NL = 16
_H = 8
_HC = 4
_DH = 16
_B = 80
_CH = 10


def _qkv_body(h_ref, w_ref, q_ref, kv_ref):
    acc = jnp.dot(h_ref[...], w_ref[...], preferred_element_type=jnp.float32)
    q = acc[:, :128] * 0.25
    q_ref[0] = q[:, :64]
    q_ref[1] = q[:, 64:]
    kv_ref[0] = jnp.concatenate([acc[:, 128:192], acc[:, 256:320]], axis=1)
    kv_ref[1] = jnp.concatenate([acc[:, 192:256], acc[:, 320:384]], axis=1)


def _qkv_call(h, w_all):
    n = h.shape[0]
    bn = 2000
    return pl.pallas_call(
        _qkv_body,
        grid=(n // bn,),
        in_specs=[
            pl.BlockSpec((bn, 128), lambda i: (i, i * 0)),
            pl.BlockSpec((128, 384), lambda i: (i * 0, i * 0)),
        ],
        out_specs=[
            pl.BlockSpec((_NC, bn, 64), lambda i: (i * 0, i, i * 0)),
            pl.BlockSpec((_NC, bn, 128), lambda i: (i * 0, i, i * 0)),
        ],
        out_shape=[
            jax.ShapeDtypeStruct((_NC, n, 64), jnp.float32),
            jax.ShapeDtypeStruct((_NC, n, 128), jnp.float32),
        ],
    )(h, w_all)


def _edge_call(q_all, kv_all, src, dst):
    n = kv_all.shape[0] // _NC
    e = src.shape[0]
    per_w = e // _NS
    chunk_e = _B * _CH
    n_chunks = per_w // chunk_e
    n2_pad = -(-((n + 1) // 2) // 128) * 128
    rows_t = n2_pad // _NS
    za_rows = -(-(-(-(n * _HC) // 128)) // 8) * 8
    n2 = n // 2

    mesh = plsc.VectorSubcoreMesh(core_axis_name="c", subcore_axis_name="s")

    def body(q_hbm, kv_hbm, src_hbm, dst_hbm, out_hbm, outz_hbm,
             acc, zsh, zacc, zbuf, raw_s, raw_d, idx_sg, idx_dg, idx_sc,
             kvr, qr, msg, gsem_kv0, gsem_kv1, gsem_q):
        gsem_kv = (gsem_kv0, gsem_kv1)
        c = lax.axis_index("c")
        s = lax.axis_index("s")
        lane = lax.iota(jnp.int32, _NL)
        zmask = lane < _HC
        zero16 = jnp.zeros((_NL,), jnp.float32)
        cn = c * n
        c7 = jnp.full((_NL,), 7, jnp.int32)
        c127 = jnp.full((_NL,), 127, jnp.int32)
        c15 = jnp.full((_NL,), 15, jnp.int32)
        one16 = jnp.full((_NL,), 1, jnp.int32)

        for rr in range(8):
            for cc in range(8):
                zbuf[rr, pl.ds(cc * _NL, _NL)] = zero16

        def zc(_, j):
            pltpu.sync_copy(zbuf, acc.at[pl.ds(s * rows_t + j * 8, 8), :])
            return j + 1
        lax.fori_loop(0, rows_t // 8, zc, jnp.int32(0))

        def zz(_, j):
            r = lax.div(j, jnp.int32(8))
            col = lax.rem(j, jnp.int32(8)) * _NL
            zacc[r, pl.ds(col, _NL)] = zero16
            return j + 1
        lax.fori_loop(0, za_rows * 8, zz, jnp.int32(0))

        @pl.when(s < 10)
        def _():
            def zs(_, j):
                pltpu.sync_copy(zbuf, zsh.at[pl.ds(s * 32 + j * 8, 8), :])
                return j + 1
            lax.fori_loop(0, 4, zs, jnp.int32(0))
        plsc.subcore_barrier()

        base0 = s * per_w

        def prep(b, k):
            def pp(_2, kk):
                sv = raw_s[pl.ds(b * _B + kk * _NL, _NL)]
                dv = raw_d[pl.ds(b * _B + kk * _NL, _NL)]
                idx_sg[k, pl.ds(kk * _NL, _NL)] = sv + cn
                dh = lax.shift_right_logical(
                    dv, jnp.full((_NL,), 1, jnp.int32))
                idx_sc[k, pl.ds(kk * _NL, _NL)] = dh
                idx_dg[k, pl.ds(kk * _NL, _NL)] = dh + c * n2
                return kk + 1
            lax.fori_loop(0, _B // _NL, pp, jnp.int32(0))
            pltpu.async_copy(kv_hbm.at[idx_sg.at[jnp.int32(k)]],
                             kvr.at[jnp.int32(k)], gsem_kv[k])

        def fire_q(k):
            pltpu.async_copy(q_hbm.at[idx_dg.at[jnp.int32(k)]], qr, gsem_q)

        def compute(b, k, fq):
            pltpu.make_async_copy(kv_hbm.at[idx_sg.at[jnp.int32(k)]],
                                  kvr.at[jnp.int32(k)], gsem_kv[k]).wait()
            pltpu.make_async_copy(q_hbm.at[idx_dg.at[jnp.int32(k)]], qr,
                                  gsem_q).wait()

            def grp(_2, kk):
                dvec = raw_d[pl.ds(b * _B + kk * _NL, _NL)]
                base_e = kk * _NL
                for j in range(_NL):
                    ei = base_e + j
                    dspl = dvec[jnp.full((_NL,), j, jnp.int32)]
                    parb = lax.bitwise_and(dspl, one16) > 0
                    z = zero16
                    for hh in range(_HC):
                        kh = kvr[k, ei, pl.ds(hh * _DH, _DH)]
                        qh0 = qr[ei, pl.ds(hh * _DH, _DH)]
                        qh1 = qr[ei, pl.ds(64 + hh * _DH, _DH)]
                        qh = jnp.where(parb, qh1, qh0)
                        cs = plsc.cumsum(kh * qh)
                        pev = jnp.exp(jnp.minimum(jnp.maximum(cs, -5.0), 5.0))
                        pe = pev[c15]
                        vh = kvr[k, ei, pl.ds(64 + hh * _DH, _DH)]
                        mh = vh * pe
                        msg[ei, pl.ds(hh * _DH, _DH)] = jnp.where(
                            parb, zero16, mh)
                        msg[ei, pl.ds(64 + hh * _DH, _DH)] = jnp.where(
                            parb, mh, zero16)
                        z = jnp.where(lane == hh, pe, z)
                    flatv = lane + dspl * _HC
                    rowv = lax.shift_right_logical(flatv, c7)
                    colv = lax.bitwise_and(flatv, c127)
                    plsc.addupdate_scatter(zacc, [rowv, colv], z, mask=zmask)
                return kk + 1
            lax.fori_loop(0, _B // _NL, grp, jnp.int32(0))
            if fq is not None:
                fire_q(fq)
            pltpu.sync_copy(msg, acc.at[idx_sc.at[jnp.int32(k)]], add=True)

        def chunk(_, ci):
            base_c = pl.multiple_of(base0 + ci * chunk_e, 8)
            pltpu.sync_copy(src_hbm.at[pl.ds(base_c, chunk_e)], raw_s)
            pltpu.sync_copy(dst_hbm.at[pl.ds(base_c, chunk_e)], raw_d)
            prep(jnp.int32(0), 0)
            fire_q(0)

            def inner(_2, io):
                b0 = io * 2
                b1 = b0 + 1
                prep(b1, 1)
                compute(b0, 0, 1)
                prep(b1 + 1, 0)
                compute(b1, 1, 0)
                return io + 1
            lax.fori_loop(0, _CH // 2 - 1, inner, jnp.int32(0))
            last = jnp.int32(_CH - 2)
            prep(last + 1, 1)
            compute(last, 0, 1)
            compute(last + 1, 1, None)
            return ci + 1
        lax.fori_loop(0, n_chunks, chunk, jnp.int32(0))

        for ch in range(za_rows // _B):
            for kk in range(_B // _NL):
                idx_sc[0, pl.ds(kk * _NL, _NL)] = lane + (ch * _B + kk * _NL)
            pltpu.sync_copy(zacc.at[pl.ds(ch * _B, _B), :],
                            zsh.at[idx_sc.at[jnp.int32(0)]], add=True)
        plsc.subcore_barrier()
        pltpu.sync_copy(acc.at[pl.ds(s * rows_t, rows_t), :],
                        out_hbm.at[c, pl.ds(s * rows_t, rows_t), :])

        @pl.when(s < 10)
        def _():
            pltpu.sync_copy(zsh.at[pl.ds(s * 32, 32), :],
                            outz_hbm.at[c, pl.ds(s * 32, 32), :])

    fn = pl.kernel(
        body,
        out_type=[
            jax.ShapeDtypeStruct((_NC, n2_pad, 128), jnp.float32),
            jax.ShapeDtypeStruct((_NC, za_rows, 128), jnp.float32),
        ],
        mesh=mesh,
        compiler_params=pltpu.CompilerParams(needs_layout_passes=False),
        scratch_types=[
            pltpu.VMEM_SHARED((n2_pad, 128), jnp.float32),
            pltpu.VMEM_SHARED((za_rows, 128), jnp.float32),
            pltpu.VMEM((za_rows, 128), jnp.float32),
            pltpu.VMEM((8, 128), jnp.float32),
            pltpu.VMEM((_B * _CH,), jnp.int32),
            pltpu.VMEM((_B * _CH,), jnp.int32),
            pltpu.VMEM((2, _B), jnp.int32),
            pltpu.VMEM((2, _B), jnp.int32),
            pltpu.VMEM((2, _B), jnp.int32),
            pltpu.VMEM((2, _B, 128), jnp.float32),
            pltpu.VMEM((_B, 128), jnp.float32),
            pltpu.VMEM((_B, 128), jnp.float32),
            pltpu.SemaphoreType.DMA,
            pltpu.SemaphoreType.DMA,
            pltpu.SemaphoreType.DMA,
        ],
    )
    wv, zp = fn(q_all, kv_all, src, dst)
    wv = wv.reshape(_NC, 2 * n2_pad, 64)
    zp = zp.reshape(_NC, za_rows * 128 // _HC, _HC)
    return wv, zp


def _ln(x, g, b):
    mu = jnp.mean(x, axis=1, keepdims=True)
    xc = x - mu
    var = jnp.mean(xc * xc, axis=1, keepdims=True)
    return xc * lax.rsqrt(var + 1e-5) * g + b


def _epi_body(h_ref, p_ref, z_ref, wo_ref, bo_ref, g1_ref, b1_ref,
              w1_ref, c1_ref, w2_ref, c2_ref, g2_ref, b2_ref, o_ref):
    wv = jnp.concatenate([p_ref[0], p_ref[1]], axis=1)
    z8 = jnp.concatenate([z_ref[0], z_ref[1]], axis=1)
    ii = lax.broadcasted_iota(jnp.int32, (_H, 128), 1)
    jj = lax.broadcasted_iota(jnp.int32, (_H, 128), 0) * _DH
    r_sel = ((ii >= jj) & (ii < jj + _DH)).astype(jnp.float32)
    zr = jnp.dot(z8, r_sel, preferred_element_type=jnp.float32)
    ha = wv / (zr + 1e-6)
    h2 = jnp.dot(ha, wo_ref[...], preferred_element_type=jnp.float32) + bo_ref[...]
    r1 = h_ref[...] + h2
    n1 = _ln(r1, g1_ref[...], b1_ref[...])
    f = jnp.dot(n1, w1_ref[...], preferred_element_type=jnp.float32) + c1_ref[...]
    f = jnp.maximum(f, 0.0)
    f = jnp.dot(f, w2_ref[...], preferred_element_type=jnp.float32) + c2_ref[...]
    r2 = n1 + f
    o_ref[...] = _ln(r2, g2_ref[...], b2_ref[...])


def _epi_call(h, wv, zp, Wo, bo, ln1_g, ln1_b, W1, b1, W2, b2, ln2_g, ln2_b):
    n = h.shape[0]
    bn = 2000
    full = lambda i: (i * 0, i * 0)
    return pl.pallas_call(
        _epi_body,
        grid=(n // bn,),
        in_specs=[
            pl.BlockSpec((bn, 128), lambda i: (i, i * 0)),
            pl.BlockSpec((_NC, bn, 64), lambda i: (i * 0, i, i * 0)),
            pl.BlockSpec((_NC, bn, _HC), lambda i: (i * 0, i, i * 0)),
            pl.BlockSpec((128, 128), full),
            pl.BlockSpec((1, 128), full),
            pl.BlockSpec((1, 128), full),
            pl.BlockSpec((1, 128), full),
            pl.BlockSpec((128, 256), full),
            pl.BlockSpec((1, 256), full),
            pl.BlockSpec((256, 128), full),
            pl.BlockSpec((1, 128), full),
            pl.BlockSpec((1, 128), full),
            pl.BlockSpec((1, 128), full),
        ],
        out_specs=pl.BlockSpec((bn, 128), lambda i: (i, i * 0)),
        out_shape=jax.ShapeDtypeStruct((n, 128), jnp.float32),
    )(h, wv, zp, Wo, bo.reshape(1, -1), ln1_g.reshape(1, -1),
      ln1_b.reshape(1, -1), W1, b1.reshape(1, -1), W2, b2.reshape(1, -1),
      ln2_g.reshape(1, -1), ln2_b.reshape(1, -1))


def kernel(h, edge_index, Wq, Wk, Wv, Wo, bo, ln1_g, ln1_b,
           W1, b1, W2, b2, ln2_g, ln2_b):
    f32 = jnp.float32
    h, Wq, Wk, Wv, Wo, bo, ln1_g, ln1_b, W1, b1, W2, b2, ln2_g, ln2_b = (
        x.astype(f32) for x in
        (h, Wq, Wk, Wv, Wo, bo, ln1_g, ln1_b, W1, b1, W2, b2, ln2_g, ln2_b))
    src = edge_index[0].astype(jnp.int32)
    dst = edge_index[1].astype(jnp.int32)
    w_all = jnp.concatenate([Wq, Wk, Wv], axis=1)
    q_all, kv_all = _qkv_call(h, w_all)
    n = h.shape[0]
    wv, zp = _edge_call(q_all.reshape(_NC * (n // 2), 128),
                        kv_all.reshape(_NC * n, 128), src, dst)
    out = _epi_call(h, wv, zp, Wo, bo, ln1_g, ln1_b,
                    W1, b1, W2, b2, ln2_g, ln2_b)
    return out.astype(jnp.float64)
